# Initial kernel scaffold; baseline (speedup 1.0000x reference)
import jax
import jax.numpy as jnp
from jax.experimental import pallas as pl


def kernel(x, perms):
    raise NotImplementedError("write your pallas kernel here")


# speedup vs baseline: 1.4379x; 1.4379x over previous
import jax
import jax.numpy as jnp
from jax import lax
from jax.experimental import pallas as pl
from jax.experimental.pallas import tpu as pltpu
from jax.experimental.pallas import tpu_sc as plsc

D = 4096
N_ROT = 4
B_ROWS = 16384
NC = 2
NS = 16
NW = NC * NS
ROWS_PER_W = B_ROWS // NW
CH = 8
N_CHUNKS = ROWS_PER_W // CH
GROUPS = D // 16


def _body(x_hbm, perms_hbm, out_hbm, perms_v, c_v, in_v, out_v):
    wid = lax.axis_index("s") * NC + lax.axis_index("c")

    pltpu.sync_copy(perms_hbm, perms_v)

    def compose(g, carry):
        base = g * 16
        i3 = perms_v[pl.ds(3 * D + base, 16)]
        i2 = plsc.load_gather(perms_v, [i3 + 2 * D])
        i1 = plsc.load_gather(perms_v, [i2 + D])
        i0 = plsc.load_gather(perms_v, [i1])
        c_v[pl.ds(base, 16)] = i0
        return carry

    lax.fori_loop(0, GROUPS, compose, None)

    row_base = wid * ROWS_PER_W

    def chunk(ci, carry):
        base = (row_base + ci * CH) * D
        pltpu.sync_copy(x_hbm.at[pl.ds(base, CH * D)], in_v)

        def per_group(g, c2):
            gb = g * 16
            idx = c_v[pl.ds(gb, 16)]
            for i in range(CH):
                vals = plsc.load_gather(in_v, [idx + i * D])
                out_v[pl.ds(i * D + gb, 16)] = vals
            return c2

        lax.fori_loop(0, GROUPS, per_group, None)
        pltpu.sync_copy(out_v, out_hbm.at[pl.ds(base, CH * D)])
        return carry

    lax.fori_loop(0, N_CHUNKS, chunk, None)


def kernel(x, perms):
    mesh = plsc.VectorSubcoreMesh(core_axis_name="c", subcore_axis_name="s")
    k = pl.kernel(
        _body,
        out_type=jax.ShapeDtypeStruct((B_ROWS * D,), jnp.float32),
        mesh=mesh,
        compiler_params=pltpu.CompilerParams(needs_layout_passes=False),
        scratch_types=[
            pltpu.VMEM((N_ROT * D,), jnp.int32),
            pltpu.VMEM((D,), jnp.int32),
            pltpu.VMEM((CH * D,), jnp.float32),
            pltpu.VMEM((CH * D,), jnp.float32),
        ],
    )
    out = k(x.reshape(-1), perms.reshape(-1))
    return out.reshape(B_ROWS, D)


# speedup vs baseline: 2.9073x; 2.0219x over previous
import jax
import jax.numpy as jnp
from jax import lax
from jax.experimental import pallas as pl
from jax.experimental.pallas import tpu as pltpu
from jax.experimental.pallas import tpu_sc as plsc

D = 4096
N_ROT = 4
B_ROWS = 16384
NC = 2
NS = 16
NW = NC * NS
ROWS_PER_W = B_ROWS // NW
CH = 4
NBUF = 2
N_CHUNKS = ROWS_PER_W // CH
GROUPS = D // 16
UNROLL = 4


def _body(x_hbm, perms_hbm, out_hbm,
          perms_v, c_v, in_v0, in_v1, out_v0, out_v1,
          in_sem0, in_sem1, out_sem0, out_sem1):
    wid = lax.axis_index("s") * NC + lax.axis_index("c")
    row_base = wid * ROWS_PER_W

    in_bufs = (in_v0, in_v1)
    out_bufs = (out_v0, out_v1)
    in_sems = (in_sem0, in_sem1)
    out_sems = (out_sem0, out_sem1)

    def copy_in(ci, b):
        base = (row_base + ci * CH) * D
        return pltpu.make_async_copy(
            x_hbm.at[pl.ds(base, CH * D)], in_bufs[b], in_sems[b])

    def copy_out(ci, b):
        base = (row_base + ci * CH) * D
        return pltpu.make_async_copy(
            out_bufs[b], out_hbm.at[pl.ds(base, CH * D)], out_sems[b])

    copy_in(0, 0).start()
    copy_in(1, 1).start()

    pltpu.sync_copy(perms_hbm, perms_v)

    @plsc.parallel_loop(0, GROUPS, 1, unroll=UNROLL)
    def compose(g):
        base = g * 16
        i3 = perms_v[pl.ds(3 * D + base, 16)]
        i2 = plsc.load_gather(perms_v, [i3 + 2 * D])
        i1 = plsc.load_gather(perms_v, [i2 + D])
        i0 = plsc.load_gather(perms_v, [i1])
        c_v[pl.ds(base, 16)] = i0

    def chunk(p, carry):
        for b in range(NBUF):
            ci = p * NBUF + b
            copy_in(ci, b).wait()

            @pl.when(ci >= NBUF)
            def _():
                copy_out(ci - NBUF, b).wait()

            in_v = in_bufs[b]
            out_v = out_bufs[b]

            @plsc.parallel_loop(0, GROUPS, 1, unroll=UNROLL)
            def per_group(g):
                gb = g * 16
                idx = c_v[pl.ds(gb, 16)]
                for i in range(CH):
                    vals = plsc.load_gather(in_v, [idx + i * D])
                    out_v[pl.ds(i * D + gb, 16)] = vals

            copy_out(ci, b).start()

            @pl.when(ci + NBUF < N_CHUNKS)
            def _():
                copy_in(ci + NBUF, b).start()
        return carry

    lax.fori_loop(0, N_CHUNKS // NBUF, chunk, None)

    copy_out(N_CHUNKS - 2, 0).wait()
    copy_out(N_CHUNKS - 1, 1).wait()


def kernel(x, perms):
    mesh = plsc.VectorSubcoreMesh(core_axis_name="c", subcore_axis_name="s")
    k = pl.kernel(
        _body,
        out_type=jax.ShapeDtypeStruct((B_ROWS * D,), jnp.float32),
        mesh=mesh,
        compiler_params=pltpu.CompilerParams(needs_layout_passes=False),
        scratch_types=[
            pltpu.VMEM((N_ROT * D,), jnp.int32),
            pltpu.VMEM((D,), jnp.int32),
            pltpu.VMEM((CH * D,), jnp.float32),
            pltpu.VMEM((CH * D,), jnp.float32),
            pltpu.VMEM((CH * D,), jnp.float32),
            pltpu.VMEM((CH * D,), jnp.float32),
            pltpu.SemaphoreType.DMA,
            pltpu.SemaphoreType.DMA,
            pltpu.SemaphoreType.DMA,
            pltpu.SemaphoreType.DMA,
        ],
    )
    out = k(x.reshape(-1), perms.reshape(-1))
    return out.reshape(B_ROWS, D)


# speedup vs baseline: 3.3146x; 1.1401x over previous
import jax
import jax.numpy as jnp
from jax import lax
from jax.experimental import pallas as pl
from jax.experimental.pallas import tpu as pltpu
from jax.experimental.pallas import tpu_sc as plsc

D = 4096
N_ROT = 4
B_ROWS = 16384
NC = 2
NS = 16
NW = NC * NS
TR_TOTAL = B_ROWS // 8
TR_PER_W = TR_TOTAL // NW
TRW = 8 * D
HALF = TRW // 2
GROUPS_HALF = D // 32
UNROLL = 4


def _body(x_hbm, perms_hbm, out_hbm,
          perms_v, c_v, in_v0, in_v1, out_v0, out_v1,
          in_sem0, in_sem1, out_sem0, out_sem1):
    wid = lax.axis_index("s") * NC + lax.axis_index("c")
    tr_base = wid * TR_PER_W

    in_bufs = (in_v0, in_v1)
    out_bufs = (out_v0, out_v1)
    in_sems = (in_sem0, in_sem1)
    out_sems = (out_sem0, out_sem1)

    def copy_in(ci, b):
        return pltpu.make_async_copy(
            x_hbm.at[tr_base + ci, :], in_bufs[b], in_sems[b])

    def copy_out(ci, h):
        return pltpu.make_async_copy(
            out_bufs[h],
            out_hbm.at[tr_base + ci, pl.ds(h * HALF, HALF)],
            out_sems[h])

    copy_in(0, 0).start()
    copy_in(1, 1).start()

    pltpu.sync_copy(perms_hbm, perms_v)

    @plsc.parallel_loop(0, D // 16, 1, unroll=UNROLL)
    def compose(g):
        base = g * 16
        i3 = perms_v[pl.ds(3 * D + base, 16)]
        i2 = plsc.load_gather(perms_v, [i3 + 2 * D])
        i1 = plsc.load_gather(perms_v, [i2 + D])
        i0 = plsc.load_gather(perms_v, [i1])
        ct = ((i0 >> 7) << 10) + (i0 & 127)
        c_v[pl.ds(base, 16)] = ct

    def chunk_pair(p, carry):
        for b in range(2):
            ci = p * 2 + b
            copy_in(ci, b).wait()
            in_v = in_bufs[b]
            for h in range(2):
                @pl.when(ci >= 1)
                def _():
                    copy_out(ci - 1, h).wait()

                out_v = out_bufs[h]

                @plsc.parallel_loop(0, GROUPS_HALF, 1, unroll=UNROLL)
                def per_group(g):
                    gb = h * HALF // 8 + g * 16
                    idx = c_v[pl.ds(gb, 16)]
                    og = (g >> 3) * 1024 + (g & 7) * 16
                    for r in range(8):
                        vals = plsc.load_gather(in_v, [idx + r * 128])
                        out_v[pl.ds(og + r * 128, 16)] = vals

                copy_out(ci, h).start()

            @pl.when(ci + 2 < TR_PER_W)
            def _():
                copy_in(ci + 2, b).start()
        return carry

    lax.fori_loop(0, TR_PER_W // 2, chunk_pair, None)

    copy_out(TR_PER_W - 1, 0).wait()
    copy_out(TR_PER_W - 1, 1).wait()


def kernel(x, perms):
    mesh = plsc.VectorSubcoreMesh(core_axis_name="c", subcore_axis_name="s")
    k = pl.kernel(
        _body,
        out_type=jax.ShapeDtypeStruct((TR_TOTAL, TRW), jnp.float32),
        mesh=mesh,
        compiler_params=pltpu.CompilerParams(needs_layout_passes=False),
        scratch_types=[
            pltpu.VMEM((N_ROT * D,), jnp.int32),
            pltpu.VMEM((D,), jnp.int32),
            pltpu.VMEM((TRW,), jnp.float32),
            pltpu.VMEM((TRW,), jnp.float32),
            pltpu.VMEM((HALF,), jnp.float32),
            pltpu.VMEM((HALF,), jnp.float32),
            pltpu.SemaphoreType.DMA,
            pltpu.SemaphoreType.DMA,
            pltpu.SemaphoreType.DMA,
            pltpu.SemaphoreType.DMA,
        ],
    )
    xv = (x.reshape(TR_TOTAL, 8, D // 128, 128)
           .transpose(0, 2, 1, 3)
           .reshape(TR_TOTAL, TRW))
    outv = k(xv, perms.reshape(-1))
    return (outv.reshape(TR_TOTAL, D // 128, 8, 128)
                .transpose(0, 2, 1, 3)
                .reshape(B_ROWS, D))


# speedup vs baseline: 8.8897x; 2.6820x over previous
import jax
import jax.numpy as jnp
from jax import lax
from jax.experimental import pallas as pl
from jax.experimental.pallas import tpu as pltpu
from jax.experimental.pallas import tpu_sc as plsc

D = 4096
N_ROT = 4
B_ROWS = 16384
NC = 2
NS = 16
NW = NC * NS
ROWS_PER_W = B_ROWS // NW
CH = 4
NBUF = 2
N_CHUNKS = ROWS_PER_W // CH
GROUPS = D // 16
UNROLL = 4


def _body(x_hbm, perms_hbm, out_hbm,
          perms_v, c_v, in_v0, in_v1, out_v0, out_v1,
          in_sem0, in_sem1, out_sem0, out_sem1):
    wid = lax.axis_index("s") * NC + lax.axis_index("c")
    row_base = wid * ROWS_PER_W

    in_bufs = (in_v0, in_v1)
    out_bufs = (out_v0, out_v1)
    in_sems = (in_sem0, in_sem1)
    out_sems = (out_sem0, out_sem1)

    def copy_in_start(ci, b):
        r0 = row_base + ci * CH
        for r in range(CH):
            pltpu.make_async_copy(
                x_hbm.at[r0 + r, :],
                in_bufs[b].at[pl.ds(r * D, D)],
                in_sems[b]).start()

    def copy_in_wait(b):
        for r in range(CH):
            pltpu.make_async_copy(
                x_hbm.at[0, :],
                in_bufs[b].at[pl.ds(r * D, D)],
                in_sems[b]).wait()

    def copy_out_start(ci, b):
        r0 = row_base + ci * CH
        for r in range(CH):
            pltpu.make_async_copy(
                out_bufs[b].at[pl.ds(r * D, D)],
                out_hbm.at[r0 + r, :],
                out_sems[b]).start()

    def copy_out_wait(b):
        for r in range(CH):
            pltpu.make_async_copy(
                out_bufs[b].at[pl.ds(r * D, D)],
                out_hbm.at[0, :],
                out_sems[b]).wait()

    copy_in_start(0, 0)
    copy_in_start(1, 1)

    for i in range(N_ROT):
        pltpu.sync_copy(perms_hbm.at[i, :], perms_v.at[pl.ds(i * D, D)])

    @plsc.parallel_loop(0, GROUPS, 1, unroll=UNROLL)
    def compose(g):
        base = g * 16
        i3 = perms_v[pl.ds(3 * D + base, 16)]
        i2 = plsc.load_gather(perms_v, [i3 + 2 * D])
        i1 = plsc.load_gather(perms_v, [i2 + D])
        i0 = plsc.load_gather(perms_v, [i1])
        c_v[pl.ds(base, 16)] = i0

    def chunk_pair(p, carry):
        for b in range(NBUF):
            ci = p * NBUF + b
            copy_in_wait(b)

            @pl.when(ci >= NBUF)
            def _():
                copy_out_wait(b)

            in_v = in_bufs[b]
            out_v = out_bufs[b]

            @plsc.parallel_loop(0, GROUPS, 1, unroll=UNROLL)
            def per_group(g):
                gb = g * 16
                idx = c_v[pl.ds(gb, 16)]
                for i in range(CH):
                    vals = plsc.load_gather(in_v, [idx + i * D])
                    out_v[pl.ds(i * D + gb, 16)] = vals

            copy_out_start(ci, b)

            @pl.when(ci + NBUF < N_CHUNKS)
            def _():
                copy_in_start(ci + NBUF, b)
        return carry

    lax.fori_loop(0, N_CHUNKS // NBUF, chunk_pair, None)

    copy_out_wait(0)
    copy_out_wait(1)


def kernel(x, perms):
    mesh = plsc.VectorSubcoreMesh(core_axis_name="c", subcore_axis_name="s")
    k = pl.kernel(
        _body,
        out_type=jax.ShapeDtypeStruct((B_ROWS, D), jnp.float32),
        mesh=mesh,
        compiler_params=pltpu.CompilerParams(needs_layout_passes=False),
        scratch_types=[
            pltpu.VMEM((N_ROT * D,), jnp.int32),
            pltpu.VMEM((D,), jnp.int32),
            pltpu.VMEM((CH * D,), jnp.float32),
            pltpu.VMEM((CH * D,), jnp.float32),
            pltpu.VMEM((CH * D,), jnp.float32),
            pltpu.VMEM((CH * D,), jnp.float32),
            pltpu.SemaphoreType.DMA,
            pltpu.SemaphoreType.DMA,
            pltpu.SemaphoreType.DMA,
            pltpu.SemaphoreType.DMA,
        ],
    )
    return k(x, perms)


# speedup vs baseline: 8.9065x; 1.0019x over previous
import jax
import jax.numpy as jnp
from jax import lax
from jax.experimental import pallas as pl
from jax.experimental.pallas import tpu as pltpu
from jax.experimental.pallas import tpu_sc as plsc

D = 4096
N_ROT = 4
B_ROWS = 16384
NC = 2
NS = 16
NW = NC * NS
ROWS_PER_W = B_ROWS // NW
CH = 4
NBUF = 2
N_CHUNKS = ROWS_PER_W // CH
GROUPS = D // 16
UNROLL = 8


def _body(x_hbm, perms_hbm, out_hbm,
          perms_v, c_v, in_v0, in_v1, out_v0, out_v1,
          in_sem0, in_sem1, out_sem0, out_sem1):
    wid = lax.axis_index("s") * NC + lax.axis_index("c")
    row_base = wid * ROWS_PER_W

    in_bufs = (in_v0, in_v1)
    out_bufs = (out_v0, out_v1)
    in_sems = (in_sem0, in_sem1)
    out_sems = (out_sem0, out_sem1)

    def copy_in_start(ci, b):
        r0 = row_base + ci * CH
        for r in range(CH):
            pltpu.make_async_copy(
                x_hbm.at[r0 + r, :],
                in_bufs[b].at[pl.ds(r * D, D)],
                in_sems[b]).start()

    def copy_in_wait(b):
        for r in range(CH):
            pltpu.make_async_copy(
                x_hbm.at[0, :],
                in_bufs[b].at[pl.ds(r * D, D)],
                in_sems[b]).wait()

    def copy_out_start(ci, b):
        r0 = row_base + ci * CH
        for r in range(CH):
            pltpu.make_async_copy(
                out_bufs[b].at[pl.ds(r * D, D)],
                out_hbm.at[r0 + r, :],
                out_sems[b]).start()

    def copy_out_wait(b):
        for r in range(CH):
            pltpu.make_async_copy(
                out_bufs[b].at[pl.ds(r * D, D)],
                out_hbm.at[0, :],
                out_sems[b]).wait()

    copy_in_start(0, 0)
    copy_in_start(1, 1)

    for i in range(N_ROT):
        pltpu.sync_copy(perms_hbm.at[i, :], perms_v.at[pl.ds(i * D, D)])

    @plsc.parallel_loop(0, GROUPS, 1, unroll=UNROLL)
    def compose(g):
        base = g * 16
        i3 = perms_v[pl.ds(3 * D + base, 16)]
        i2 = plsc.load_gather(perms_v, [i3 + 2 * D])
        i1 = plsc.load_gather(perms_v, [i2 + D])
        i0 = plsc.load_gather(perms_v, [i1])
        c_v[pl.ds(base, 16)] = i0

    def chunk_pair(p, carry):
        for b in range(NBUF):
            ci = p * NBUF + b
            copy_in_wait(b)

            @pl.when(ci >= NBUF)
            def _():
                copy_out_wait(b)

            in_v = in_bufs[b]
            out_v = out_bufs[b]

            @plsc.parallel_loop(0, GROUPS, 1, unroll=UNROLL)
            def per_group(g):
                gb = g * 16
                idx = c_v[pl.ds(gb, 16)]
                for i in range(CH):
                    vals = plsc.load_gather(in_v, [idx + i * D])
                    out_v[pl.ds(i * D + gb, 16)] = vals

            copy_out_start(ci, b)

            @pl.when(ci + NBUF < N_CHUNKS)
            def _():
                copy_in_start(ci + NBUF, b)
        return carry

    lax.fori_loop(0, N_CHUNKS // NBUF, chunk_pair, None)

    copy_out_wait(0)
    copy_out_wait(1)


def kernel(x, perms):
    mesh = plsc.VectorSubcoreMesh(core_axis_name="c", subcore_axis_name="s")
    k = pl.kernel(
        _body,
        out_type=jax.ShapeDtypeStruct((B_ROWS, D), jnp.float32),
        mesh=mesh,
        compiler_params=pltpu.CompilerParams(needs_layout_passes=False),
        scratch_types=[
            pltpu.VMEM((N_ROT * D,), jnp.int32),
            pltpu.VMEM((D,), jnp.int32),
            pltpu.VMEM((CH * D,), jnp.float32),
            pltpu.VMEM((CH * D,), jnp.float32),
            pltpu.VMEM((CH * D,), jnp.float32),
            pltpu.VMEM((CH * D,), jnp.float32),
            pltpu.SemaphoreType.DMA,
            pltpu.SemaphoreType.DMA,
            pltpu.SemaphoreType.DMA,
            pltpu.SemaphoreType.DMA,
        ],
    )
    return k(x, perms)


# speedup vs baseline: 9.1293x; 1.0250x over previous
import jax
import jax.numpy as jnp
from jax import lax
from jax.experimental import pallas as pl
from jax.experimental.pallas import tpu as pltpu
from jax.experimental.pallas import tpu_sc as plsc

D = 4096
N_ROT = 4
B_ROWS = 16384
NC = 2
NS = 16
NW = NC * NS
ROWS_PER_W = B_ROWS // NW
CH = 8
HR = CH // 2
NBUF = 2
N_CHUNKS = ROWS_PER_W // CH
GROUPS = D // 16
UNROLL = 4


def _body(x_hbm, perms_hbm, out_hbm,
          perms_v, c_v, in_v0, in_v1, out_v0, out_v1,
          in_sem0, in_sem1, out_sem0, out_sem1):
    wid = lax.axis_index("s") * NC + lax.axis_index("c")
    row_base = wid * ROWS_PER_W

    in_bufs = (in_v0, in_v1)
    out_bufs = (out_v0, out_v1)
    in_sems = (in_sem0, in_sem1)
    out_sems = (out_sem0, out_sem1)

    def copy_in_start(ci, b):
        r0 = row_base + ci * CH
        for r in range(CH):
            pltpu.make_async_copy(
                x_hbm.at[r0 + r, :],
                in_bufs[b].at[pl.ds(r * D, D)],
                in_sems[b]).start()

    def copy_in_wait(b):
        for r in range(CH):
            pltpu.make_async_copy(
                x_hbm.at[0, :],
                in_bufs[b].at[pl.ds(r * D, D)],
                in_sems[b]).wait()

    def copy_out_start(ci, h):
        r0 = row_base + ci * CH + h * HR
        for r in range(HR):
            pltpu.make_async_copy(
                out_bufs[h].at[pl.ds(r * D, D)],
                out_hbm.at[r0 + r, :],
                out_sems[h]).start()

    def copy_out_wait(h):
        for r in range(HR):
            pltpu.make_async_copy(
                out_bufs[h].at[pl.ds(r * D, D)],
                out_hbm.at[0, :],
                out_sems[h]).wait()

    copy_in_start(0, 0)
    copy_in_start(1, 1)

    for i in range(N_ROT):
        pltpu.sync_copy(perms_hbm.at[i, :], perms_v.at[pl.ds(i * D, D)])

    @plsc.parallel_loop(0, GROUPS, 1, unroll=UNROLL)
    def compose(g):
        base = g * 16
        i3 = perms_v[pl.ds(3 * D + base, 16)]
        i2 = plsc.load_gather(perms_v, [i3 + 2 * D])
        i1 = plsc.load_gather(perms_v, [i2 + D])
        i0 = plsc.load_gather(perms_v, [i1])
        c_v[pl.ds(base, 16)] = i0

    def chunk_pair(p, carry):
        for b in range(NBUF):
            ci = p * NBUF + b
            copy_in_wait(b)
            in_v = in_bufs[b]
            for h in range(2):
                @pl.when(ci >= 1)
                def _():
                    copy_out_wait(h)

                out_v = out_bufs[h]

                @plsc.parallel_loop(0, GROUPS, 1, unroll=UNROLL)
                def per_group(g):
                    gb = g * 16
                    idx = c_v[pl.ds(gb, 16)]
                    for i in range(HR):
                        vals = plsc.load_gather(
                            in_v, [idx + (h * HR + i) * D])
                        out_v[pl.ds(i * D + gb, 16)] = vals

                copy_out_start(ci, h)

            @pl.when(ci + NBUF < N_CHUNKS)
            def _():
                copy_in_start(ci + NBUF, b)
        return carry

    lax.fori_loop(0, N_CHUNKS // NBUF, chunk_pair, None)

    copy_out_wait(0)
    copy_out_wait(1)


def kernel(x, perms):
    mesh = plsc.VectorSubcoreMesh(core_axis_name="c", subcore_axis_name="s")
    k = pl.kernel(
        _body,
        out_type=jax.ShapeDtypeStruct((B_ROWS, D), jnp.float32),
        mesh=mesh,
        compiler_params=pltpu.CompilerParams(needs_layout_passes=False),
        scratch_types=[
            pltpu.VMEM((N_ROT * D,), jnp.int32),
            pltpu.VMEM((D,), jnp.int32),
            pltpu.VMEM((CH * D,), jnp.float32),
            pltpu.VMEM((CH * D,), jnp.float32),
            pltpu.VMEM((HR * D,), jnp.float32),
            pltpu.VMEM((HR * D,), jnp.float32),
            pltpu.SemaphoreType.DMA,
            pltpu.SemaphoreType.DMA,
            pltpu.SemaphoreType.DMA,
            pltpu.SemaphoreType.DMA,
        ],
    )
    return k(x, perms)
